# Initial kernel scaffold; baseline (speedup 1.0000x reference)
import jax
import jax.numpy as jnp
from jax.experimental import pallas as pl


def kernel(social_occ, ego_state_raw, nbr_state_raw_grid, ego_lane, nbr_lane_grid, nbr_dist_grid):
    raise NotImplementedError("write your pallas kernel here")


# speedup vs baseline: 1.1907x; 1.1907x over previous
import functools

import jax
import jax.numpy as jnp
from jax import lax
from jax.experimental import pallas as pl
from jax.experimental.pallas import tpu as pltpu

_B = 4096
_N = 128
_TOPK = 6
_DIST_THRESH = 120.0
_BB = 512


def _body(occ_ref, x_ref, y_ref, v_ref, lane_ref, dist_ref,
          ex_ref, ey_ref, ev_ref, el_ref, score_out, idx_out):
    occ = occ_ref[...] > 0.5
    x = x_ref[...]
    y = y_ref[...]
    v = v_ref[...]
    lane = lane_ref[...]
    dist = dist_ref[...]
    ex = ex_ref[...]
    ey = ey_ref[...]
    ev = ev_ref[...]
    el = el_ref[...]

    lane_delta = lane - el
    same_lane = jnp.where(jnp.abs(lane_delta) < 0.5, 0.2, 0.0)
    adj_lane = jnp.where(jnp.abs(jnp.abs(lane_delta) - 1.0) < 0.5, 0.1, 0.0)
    dx = jnp.abs(x - ex)
    dy = jnp.abs(y - ey)
    closing = jnp.maximum(ev - v, 0.0)
    score = (1.25 / (dy + 1.0) + 0.75 / (dist + 1.0)
             + 0.25 * jnp.minimum(closing * 0.1, 2.0)
             + same_lane + adj_lane + 0.15 / (dx + 1.0))
    neg_inf = jnp.float32(-jnp.inf)
    close = dist <= _DIST_THRESH
    has_close = jnp.any(occ & close, axis=1, keepdims=True)
    avail = occ & (jnp.logical_not(has_close) | close)
    ms = jnp.where(avail, score, neg_inf)

    iota = lax.broadcasted_iota(jnp.int32, (_BB, _N), 1)
    taken = jnp.zeros((_BB, _N), jnp.bool_)
    score_cols = []
    idx_cols = []
    for _ in range(_TOPK):
        sc_eff = jnp.where(taken, neg_inf, ms)
        m = jnp.max(sc_eff, axis=1, keepdims=True)
        cand = jnp.where((ms == m) & jnp.logical_not(taken), iota, _N)
        idx = jnp.min(cand, axis=1, keepdims=True)
        score_cols.append(m)
        idx_cols.append(idx)
        taken = taken | (iota == idx)
    score_out[...] = jnp.concatenate(score_cols, axis=1)
    idx_out[...] = jnp.concatenate(idx_cols, axis=1)


def kernel(social_occ, ego_state_raw, nbr_state_raw_grid, ego_lane,
           nbr_lane_grid, nbr_dist_grid):
    occf = social_occ.astype(jnp.float32)
    nbr_last = nbr_state_raw_grid[:, :, -1, :]
    x = nbr_last[:, :, 0]
    y = nbr_last[:, :, 1]
    v = nbr_last[:, :, 2]
    lane = nbr_lane_grid[:, :, -1, 0]
    dist = nbr_dist_grid[:, :, -1, 0]
    ego_last = ego_state_raw[:, -1, :]
    ex = ego_last[:, 0:1]
    ey = ego_last[:, 1:2]
    ev = ego_last[:, 2:3]
    el = ego_lane[:, -1, 0:1]

    wide = pl.BlockSpec((_BB, _N), lambda i: (i, 0))
    narrow = pl.BlockSpec((_BB, 1), lambda i: (i, 0))
    out_spec = pl.BlockSpec((_BB, _TOPK), lambda i: (i, 0))
    topk_score, selected_idx = pl.pallas_call(
        _body,
        grid=(_B // _BB,),
        in_specs=[wide, wide, wide, wide, wide, wide,
                  narrow, narrow, narrow, narrow],
        out_specs=[out_spec, out_spec],
        out_shape=[jax.ShapeDtypeStruct((_B, _TOPK), jnp.float32),
                   jax.ShapeDtypeStruct((_B, _TOPK), jnp.int32)],
    )(occf, x, y, v, lane, dist, ex, ey, ev, el)
    selected_valid = jnp.isfinite(topk_score)
    return topk_score, selected_idx, selected_valid
